# baseline (speedup 1.0000x reference)
import functools

import jax
import jax.numpy as jnp
from jax import lax
from jax.experimental import pallas as pl
from jax.experimental.pallas import tpu as pltpu
from jax.experimental.pallas import tpu_sc as plsc

_NUM_CORES = 2
_NUM_SUBCORES = 16
_NUM_WORKERS = _NUM_CORES * _NUM_SUBCORES
_CHUNK = 128


def _emb_kernel(n_chunks, d, tok_hbm, w_hbm, out_hbm, idx_v, rows_v, gsem):
    wid = lax.axis_index("s") * _NUM_CORES + lax.axis_index("c")
    base = wid * (n_chunks * _CHUNK)
    pltpu.sync_copy(tok_hbm.at[wid], idx_v)

    def body(j, carry):
        pltpu.async_copy(w_hbm.at[idx_v.at[j]], rows_v, gsem).wait()
        pltpu.sync_copy(rows_v, out_hbm.at[pl.ds(base + j * _CHUNK, _CHUNK)])
        return carry

    lax.fori_loop(0, n_chunks, body, 0)


def kernel(token_ids, weight):
    b, l = token_ids.shape
    v, d = weight.shape
    n = b * l
    per_w = n // _NUM_WORKERS
    n_chunks = per_w // _CHUNK
    assert per_w * _NUM_WORKERS == n and n_chunks * _CHUNK == per_w

    tok = token_ids.reshape(_NUM_WORKERS, n_chunks, _CHUNK).astype(jnp.int32)
    mesh = plsc.VectorSubcoreMesh(
        core_axis_name="c",
        subcore_axis_name="s",
        num_cores=_NUM_CORES,
        num_subcores=_NUM_SUBCORES,
    )
    run = functools.partial(
        pl.kernel,
        mesh=mesh,
        compiler_params=pltpu.CompilerParams(use_tc_tiling_on_sc=False),
        out_type=jax.ShapeDtypeStruct((n, d), jnp.float32),
        scratch_types=[
            pltpu.VMEM((n_chunks, _CHUNK), jnp.int32),
            pltpu.VMEM((_CHUNK, d), jnp.float32),
            pltpu.SemaphoreType.DMA,
        ],
    )(functools.partial(_emb_kernel, n_chunks, d))
    out = run(tok, weight)
    return out.reshape(b, l, d)


# speedup vs baseline: 1.0450x; 1.0450x over previous
import functools

import jax
import jax.numpy as jnp
from jax import lax
from jax.experimental import pallas as pl
from jax.experimental.pallas import tpu as pltpu
from jax.experimental.pallas import tpu_sc as plsc

_NUM_CORES = 2
_NUM_SUBCORES = 16
_NUM_WORKERS = _NUM_CORES * _NUM_SUBCORES
_CHUNK = 128
_K = 5


def _emb_kernel(n_chunks, d, tok_hbm, w_hbm, out_hbm, idx_v, rows_v, g0, g1, s0, s1):
    wid = lax.axis_index("s") * _NUM_CORES + lax.axis_index("c")
    n_groups = n_chunks // _K
    base = wid * n_chunks
    gsem = (g0, g1)
    ssem = (s0, s1)
    pltpu.sync_copy(tok_hbm.at[wid], idx_v)

    def fire_group(j, b):
        return [
            pltpu.async_copy(
                w_hbm.at[idx_v.at[j * _K + i]], rows_v.at[b, i], gsem[b]
            )
            for i in range(_K)
        ]

    gathers = {}
    stores = {}
    gathers[0] = fire_group(0, 0)
    for j in range(n_groups):
        b = j % 2
        for c in gathers[j]:
            c.wait()
        stores[j] = pltpu.async_copy(
            rows_v.at[b], out_hbm.at[pl.ds(base + j * _K, _K)], ssem[b]
        )
        if j >= 1:
            stores[j - 1].wait()
        if j + 1 < n_groups:
            gathers[j + 1] = fire_group(j + 1, 1 - b)
    stores[n_groups - 1].wait()


def kernel(token_ids, weight):
    b, l = token_ids.shape
    v, d = weight.shape
    n = b * l
    per_w = n // _NUM_WORKERS
    n_chunks = per_w // _CHUNK
    assert per_w * _NUM_WORKERS == n and n_chunks * _CHUNK == per_w
    assert n_chunks % _K == 0

    tok = token_ids.reshape(_NUM_WORKERS, n_chunks, _CHUNK).astype(jnp.int32)
    mesh = plsc.VectorSubcoreMesh(
        core_axis_name="c",
        subcore_axis_name="s",
        num_cores=_NUM_CORES,
        num_subcores=_NUM_SUBCORES,
    )
    run = functools.partial(
        pl.kernel,
        mesh=mesh,
        compiler_params=pltpu.CompilerParams(use_tc_tiling_on_sc=False),
        out_type=jax.ShapeDtypeStruct((n // _CHUNK, _CHUNK, d), jnp.float32),
        scratch_types=[
            pltpu.VMEM((n_chunks, _CHUNK), jnp.int32),
            pltpu.VMEM((2, _K, _CHUNK, d), jnp.float32),
            pltpu.SemaphoreType.DMA,
            pltpu.SemaphoreType.DMA,
            pltpu.SemaphoreType.DMA,
            pltpu.SemaphoreType.DMA,
        ],
    )(functools.partial(_emb_kernel, n_chunks, d))
    out = run(tok, weight)
    return out.reshape(b, l, d)
